# baseline (speedup 1.0000x reference)
import jax
import jax.numpy as jnp
from jax import lax
from jax.experimental import pallas as pl
from jax.experimental.pallas import tpu as pltpu
from jax.experimental.pallas import tpu_sc as plsc

_B, _L, _C = 32, 2101, 64
_S = 64
_NB = 32
_W = _NB * _S
_PAD = _L - _W
_NC, _NS = 2, 16
_CHUNK = 128
_NCHUNK = _W // _CHUNK
_Q = _C // 16


def _block_sum(obuf, ybuf, slot, base, nrows):
    unroll = 8
    def body(r, acc):
        new = []
        for u in range(unroll):
            row = base + unroll * r + u
            for q in range(_Q):
                o = obuf[slot, row, pl.ds(q * 16, 16)]
                y = ybuf[slot, row, pl.ds(q * 16, 16)]
                d = o - y
                new.append(d * d)
        out = list(acc)
        for u in range(unroll):
            for q in range(_Q):
                out[q] = out[q] + new[u * _Q + q]
        return tuple(out)

    zero = (jnp.zeros((16,), jnp.float32),) * _Q
    acc = lax.fori_loop(0, nrows // unroll, body, zero)
    for row in range(base + (nrows // unroll) * unroll, base + nrows):
        out = []
        for q in range(_Q):
            o = obuf[slot, row, pl.ds(q * 16, 16)]
            y = ybuf[slot, row, pl.ds(q * 16, 16)]
            d = o - y
            out.append(acc[q] + d * d)
        acc = tuple(out)
    return acc


def _vadd(a, b):
    return tuple(x + y for x, y in zip(a, b))


def _sc_body(o_hbm, y_hbm, out_hbm, obuf, ybuf, vbuf, isem, osem):
    cid = lax.axis_index("c")
    sid = lax.axis_index("s")
    b = sid * _NC + cid

    def in_copies(k, slot):
        rows = _CHUNK if k < _NCHUNK else _PAD
        src_o = o_hbm.at[b, pl.ds(k * _CHUNK, rows), :]
        src_y = y_hbm.at[b, pl.ds(k * _CHUNK, rows), :]
        if rows == _CHUNK:
            dst_o, dst_y = obuf.at[slot], ybuf.at[slot]
        else:
            dst_o = obuf.at[slot, pl.ds(0, rows)]
            dst_y = ybuf.at[slot, pl.ds(0, rows)]
        return (pltpu.make_async_copy(src_o, dst_o, isem.at[slot, 0]),
                pltpu.make_async_copy(src_y, dst_y, isem.at[slot, 1]))

    def out_copy(p, slot, rows=_CHUNK):
        if rows == _CHUNK:
            src = vbuf.at[slot]
        else:
            src = vbuf.at[slot, pl.ds(0, rows)]
        return pltpu.make_async_copy(
            src, out_hbm.at[b, pl.ds(p * _CHUNK, rows), :], osem.at[slot])

    def fill_pair(slot, vL, vR):
        def body(r, carry, _slot=slot, _vL=vL, _vR=vR):
            for u in range(4):
                for q in range(_Q):
                    vbuf[_slot, 4 * r + u, pl.ds(q * 16, 16)] = _vL[q]
                    vbuf[_slot, _S + 4 * r + u, pl.ds(q * 16, 16)] = _vR[q]
            return carry
        lax.fori_loop(0, _S // 4, body, 0)

    def fill_tail(slot, tv):
        def body(r, carry, _slot=slot, _tv=tv):
            for q in range(_Q):
                vbuf[_slot, r, pl.ds(q * 16, 16)] = _tv[q]
            return carry
        lax.fori_loop(0, _PAD, body, 0)

    def sign(v):
        return tuple(jnp.sign(x) for x in v)

    def div(a, c):
        return tuple(x / y for x, y in zip(a, c))

    for cp in in_copies(0, 0):
        cp.start()
    s_m3 = s_m2 = s_m1 = None
    a0 = a1 = None
    for k in range(_NCHUNK + 1):
        slot = k % 2
        for cp in in_copies(k, slot):
            cp.wait()
        if k + 1 <= _NCHUNK:
            for cp in in_copies(k + 1, 1 - slot):
                cp.start()
        if k < _NCHUNK:
            a0 = _block_sum(obuf, ybuf, slot, 0, _S)
            a1 = _block_sum(obuf, ybuf, slot, _S, _S)
        else:
            tail_v = _block_sum(obuf, ybuf, slot, 0, _PAD)
        if k >= 1:
            p = k - 1
            oslot = p % 2
            if p >= 2:
                out_copy(p - 2, oslot).wait()
            if p == 0:
                mp0 = _vadd(s_m2, s_m1)
                mp1 = _vadd(s_m1, a0)
                vL = div(mp0, sign(mp0))
                vR = div(_vadd(mp0, mp1), _vadd(sign(mp0), sign(mp1)))
            elif k < _NCHUNK:
                mpa = _vadd(s_m3, s_m2)
                mpb = _vadd(s_m2, s_m1)
                mpc = _vadd(s_m1, a0)
                vL = div(_vadd(mpa, mpb), _vadd(sign(mpa), sign(mpb)))
                vR = div(_vadd(mpb, mpc), _vadd(sign(mpb), sign(mpc)))
            else:
                mpa = _vadd(s_m3, s_m2)
                mpb = _vadd(s_m2, s_m1)
                vL = div(_vadd(mpa, mpb), _vadd(sign(mpa), sign(mpb)))
                vR = div(mpb, sign(mpb))
            fill_pair(oslot, vL, vR)
            out_copy(p, oslot).start()
        if k < _NCHUNK:
            s_m3, s_m2, s_m1 = (s_m1, a0, a1) if k > 0 else (None, a0, a1)

    tslot = _NCHUNK % 2
    out_copy(_NCHUNK - 2, tslot).wait()
    fill_tail(tslot, tail_v)
    out_copy(_NCHUNK, tslot, rows=_PAD).start()
    out_copy(_NCHUNK - 1, (_NCHUNK - 1) % 2).wait()
    out_copy(_NCHUNK, tslot, rows=_PAD).wait()


def kernel(outputs, batch_y):
    mesh = plsc.VectorSubcoreMesh(core_axis_name="c", subcore_axis_name="s",
                                  num_cores=_NC, num_subcores=_NS)
    run = pl.kernel(
        _sc_body,
        out_type=jax.ShapeDtypeStruct((_B, _L, _C), jnp.float32),
        mesh=mesh,
        scratch_types=[
            pltpu.VMEM((2, _CHUNK, _C), jnp.float32),
            pltpu.VMEM((2, _CHUNK, _C), jnp.float32),
            pltpu.VMEM((2, _CHUNK, _C), jnp.float32),
            pltpu.SemaphoreType.DMA((2, 2)),
            pltpu.SemaphoreType.DMA((2,)),
        ],
    )
    return run(outputs, batch_y)


# speedup vs baseline: 1.0235x; 1.0235x over previous
import jax
import jax.numpy as jnp
from jax import lax
from jax.experimental import pallas as pl
from jax.experimental.pallas import tpu as pltpu
from jax.experimental.pallas import tpu_sc as plsc

_B, _L, _C = 32, 2101, 64
_S = 64
_NB = 32
_W = _NB * _S
_PAD = _L - _W
_NC, _NS = 2, 16
_CHUNK = 128
_NCHUNK = _W // _CHUNK
_Q = _C // 16


def _block_sum(obuf, ybuf, slot, base, nrows):
    unroll = 4
    def body(r, acc):
        new = []
        for u in range(unroll):
            row = base + unroll * r + u
            for q in range(_Q):
                o = obuf[slot, row, pl.ds(q * 16, 16)]
                y = ybuf[slot, row, pl.ds(q * 16, 16)]
                d = o - y
                new.append(d * d)
        out = list(acc)
        for u in range(unroll):
            for q in range(_Q):
                out[q] = out[q] + new[u * _Q + q]
        return tuple(out)

    zero = (jnp.zeros((16,), jnp.float32),) * _Q
    acc = lax.fori_loop(0, nrows // unroll, body, zero)
    for row in range(base + (nrows // unroll) * unroll, base + nrows):
        out = []
        for q in range(_Q):
            o = obuf[slot, row, pl.ds(q * 16, 16)]
            y = ybuf[slot, row, pl.ds(q * 16, 16)]
            d = o - y
            out.append(acc[q] + d * d)
        acc = tuple(out)
    return acc


def _vadd(a, b):
    return tuple(x + y for x, y in zip(a, b))


def _sc_body(o_hbm, y_hbm, out_hbm, obuf, ybuf, vbuf, isem, osem):
    cid = lax.axis_index("c")
    sid = lax.axis_index("s")
    b = sid * _NC + cid

    def in_copies(k, slot):
        rows = _CHUNK if k < _NCHUNK else _PAD
        src_o = o_hbm.at[b, pl.ds(k * _CHUNK, rows), :]
        src_y = y_hbm.at[b, pl.ds(k * _CHUNK, rows), :]
        if rows == _CHUNK:
            dst_o, dst_y = obuf.at[slot], ybuf.at[slot]
        else:
            dst_o = obuf.at[slot, pl.ds(0, rows)]
            dst_y = ybuf.at[slot, pl.ds(0, rows)]
        return (pltpu.make_async_copy(src_o, dst_o, isem.at[slot, 0]),
                pltpu.make_async_copy(src_y, dst_y, isem.at[slot, 1]))

    def out_copy(p, slot, rows=_CHUNK):
        if rows == _CHUNK:
            src = vbuf.at[slot]
        else:
            src = vbuf.at[slot, pl.ds(0, rows)]
        return pltpu.make_async_copy(
            src, out_hbm.at[b, pl.ds(p * _CHUNK, rows), :], osem.at[slot])

    def fill_pair(slot, vL, vR):
        def body(r, carry, _slot=slot, _vL=vL, _vR=vR):
            for u in range(4):
                for q in range(_Q):
                    vbuf[_slot, 4 * r + u, pl.ds(q * 16, 16)] = _vL[q]
                    vbuf[_slot, _S + 4 * r + u, pl.ds(q * 16, 16)] = _vR[q]
            return carry
        lax.fori_loop(0, _S // 4, body, 0)

    def fill_tail(slot, tv):
        def body(r, carry, _slot=slot, _tv=tv):
            for q in range(_Q):
                vbuf[_slot, r, pl.ds(q * 16, 16)] = _tv[q]
            return carry
        lax.fori_loop(0, _PAD, body, 0)

    def sign(v):
        return tuple(jnp.sign(x) for x in v)

    def div(a, c):
        return tuple(x / y for x, y in zip(a, c))

    for cp in in_copies(0, 0):
        cp.start()
    s_m3 = s_m2 = s_m1 = None
    a0 = a1 = None
    for k in range(_NCHUNK + 1):
        slot = k % 2
        for cp in in_copies(k, slot):
            cp.wait()
        if k + 1 <= _NCHUNK:
            for cp in in_copies(k + 1, 1 - slot):
                cp.start()
        if k < _NCHUNK:
            a0 = _block_sum(obuf, ybuf, slot, 0, _S)
            a1 = _block_sum(obuf, ybuf, slot, _S, _S)
        else:
            tail_v = _block_sum(obuf, ybuf, slot, 0, _PAD)
        if k >= 1:
            p = k - 1
            oslot = p % 2
            if p >= 2:
                out_copy(p - 2, oslot).wait()
            if p == 0:
                mp0 = _vadd(s_m2, s_m1)
                mp1 = _vadd(s_m1, a0)
                vL = div(mp0, sign(mp0))
                vR = div(_vadd(mp0, mp1), _vadd(sign(mp0), sign(mp1)))
            elif k < _NCHUNK:
                mpa = _vadd(s_m3, s_m2)
                mpb = _vadd(s_m2, s_m1)
                mpc = _vadd(s_m1, a0)
                vL = div(_vadd(mpa, mpb), _vadd(sign(mpa), sign(mpb)))
                vR = div(_vadd(mpb, mpc), _vadd(sign(mpb), sign(mpc)))
            else:
                mpa = _vadd(s_m3, s_m2)
                mpb = _vadd(s_m2, s_m1)
                vL = div(_vadd(mpa, mpb), _vadd(sign(mpa), sign(mpb)))
                vR = div(mpb, sign(mpb))
            fill_pair(oslot, vL, vR)
            out_copy(p, oslot).start()
        if k < _NCHUNK:
            s_m3, s_m2, s_m1 = (s_m1, a0, a1) if k > 0 else (None, a0, a1)

    tslot = _NCHUNK % 2
    out_copy(_NCHUNK - 2, tslot).wait()
    fill_tail(tslot, tail_v)
    out_copy(_NCHUNK, tslot, rows=_PAD).start()
    out_copy(_NCHUNK - 1, (_NCHUNK - 1) % 2).wait()
    out_copy(_NCHUNK, tslot, rows=_PAD).wait()


def kernel(outputs, batch_y):
    mesh = plsc.VectorSubcoreMesh(core_axis_name="c", subcore_axis_name="s",
                                  num_cores=_NC, num_subcores=_NS)
    run = pl.kernel(
        _sc_body,
        out_type=jax.ShapeDtypeStruct((_B, _L, _C), jnp.float32),
        mesh=mesh,
        scratch_types=[
            pltpu.VMEM((2, _CHUNK, _C), jnp.float32),
            pltpu.VMEM((2, _CHUNK, _C), jnp.float32),
            pltpu.VMEM((2, _CHUNK, _C), jnp.float32),
            pltpu.SemaphoreType.DMA((2, 2)),
            pltpu.SemaphoreType.DMA((2,)),
        ],
    )
    return run(outputs, batch_y)
